# baseline (speedup 1.0000x reference)
import functools

import jax
import jax.numpy as jnp
from jax import lax
from jax.experimental import pallas as pl
from jax.experimental.pallas import tpu as pltpu
from jax.experimental.pallas import tpu_sc as plsc

TOKENS = 8192
DIM = 2048
NUM_EXPERTS = 64
TOP_K = 8
TEMPERATURE = 0.5
BLOCK_T = 1024

_NC = 2
_NS = 16
_NW = _NC * _NS
RPW = TOKENS // _NW
NG = RPW // 16

_SORT8 = [(0, 1), (2, 3), (4, 5), (6, 7),
          (0, 2), (1, 3), (4, 6), (5, 7),
          (1, 2), (5, 6), (0, 4), (3, 7),
          (1, 5), (2, 6),
          (1, 4), (3, 6),
          (2, 4), (3, 5),
          (3, 4)]
_BITONIC8 = [(0, 4), (1, 5), (2, 6), (3, 7),
             (0, 2), (1, 3), (4, 6), (5, 7),
             (0, 1), (2, 3), (4, 5), (6, 7)]


def _logits_block(g_ref, w_ref, b_ref, alpha_ref, eb_ref, out_ref):
    w = w_ref[...]
    g = g_ref[...]
    scale = alpha_ref[0] / TEMPERATURE
    mm = lax.dot_general(w, g, (((1,), (1,)), ((), ())),
                         preferred_element_type=jnp.float32)
    out_ref[...] = (mm + b_ref[...]) * scale + eb_ref[...]


def _tc_logits(g_emb, W, b2, alpha1, eb2):
    return pl.pallas_call(
        _logits_block,
        grid=(TOKENS // BLOCK_T,),
        in_specs=[
            pl.BlockSpec((BLOCK_T, DIM), lambda i: (i, 0)),
            pl.BlockSpec((NUM_EXPERTS, DIM), lambda i: (0, 0)),
            pl.BlockSpec((NUM_EXPERTS, 1), lambda i: (0, 0)),
            pl.BlockSpec(memory_space=pltpu.SMEM),
            pl.BlockSpec((NUM_EXPERTS, 1), lambda i: (0, 0)),
        ],
        out_specs=pl.BlockSpec((NUM_EXPERTS, BLOCK_T), lambda i: (0, i)),
        out_shape=jax.ShapeDtypeStruct((NUM_EXPERTS, TOKENS), jnp.float32),
    )(g_emb, W, b2, alpha1, eb2)


def _ce_desc(lst, i, j):
    hi = jnp.maximum(lst[i], lst[j])
    lo = jnp.minimum(lst[i], lst[j])
    lst[i] = hi
    lst[j] = lo


def _merge_top8(a, b, clean):
    t = [jnp.maximum(a[i], b[7 - i]) for i in range(8)]
    if clean:
        for i, j in _BITONIC8:
            _ce_desc(t, i, j)
    return t


def _route_body(lt_hbm, out_hbm, in_v, out_v, em_v):
    wid = lax.axis_index("s") * _NC + lax.axis_index("c")
    base = wid * RPW
    pltpu.sync_copy(lt_hbm.at[:, pl.ds(base, RPW)], in_v)

    def group(gi, carry):
        col = gi * 16
        v = [in_v[e, pl.ds(col, 16)] for e in range(NUM_EXPERTS)]

        sgs = []
        for g8 in range(8):
            grp = [v[g8 * 8 + i] for i in range(8)]
            for i, j in _SORT8:
                _ce_desc(grp, i, j)
            sgs.append(grp)
        l1 = [_merge_top8(sgs[2 * i], sgs[2 * i + 1], True) for i in range(4)]
        l2 = [_merge_top8(l1[0], l1[1], True), _merge_top8(l1[2], l1[3], True)]
        t = _merge_top8(l2[0], l2[1], False)
        thr = t[0]
        mx = t[0]
        for i in range(1, 8):
            thr = jnp.minimum(thr, t[i])
            mx = jnp.maximum(mx, t[i])

        acc_all = [jnp.zeros((16,), jnp.float32) for _ in range(4)]
        acc_sel = [jnp.zeros((16,), jnp.float32) for _ in range(4)]
        for e in range(NUM_EXPERTS):
            o = in_v[e, pl.ds(col, 16)]
            ev = jnp.exp(o - mx)
            eme = jnp.where(o >= thr, ev, 0.0)
            em_v[pl.ds(e * 16, 16)] = eme
            acc_all[e % 4] = acc_all[e % 4] + ev
            acc_sel[e % 4] = acc_sel[e % 4] + eme
        s_all = (acc_all[0] + acc_all[1]) + (acc_all[2] + acc_all[3])
        s_sel = (acc_sel[0] + acc_sel[1]) + (acc_sel[2] + acc_sel[3])
        inv = 1.0 / (s_sel + 1e-12 * s_all)

        for e in range(NUM_EXPERTS):
            sl = pl.ds(e * 16, 16)
            em_v[sl] = em_v[sl] * inv
        stride16 = lax.iota(jnp.int32, 16) * 16
        for i in range(16):
            for j in range(4):
                wtok = plsc.load_gather(em_v, [stride16 + (256 * j + i)])
                out_v[pl.ds((col + i) * NUM_EXPERTS + 16 * j, 16)] = wtok
        return carry

    lax.fori_loop(0, NG, group, 0)
    pltpu.sync_copy(out_v, out_hbm.at[pl.ds(base * NUM_EXPERTS, RPW * NUM_EXPERTS)])


_route_sc = pl.kernel(
    _route_body,
    out_type=jax.ShapeDtypeStruct((TOKENS * NUM_EXPERTS,), jnp.float32),
    mesh=plsc.VectorSubcoreMesh(core_axis_name="c", subcore_axis_name="s",
                                num_cores=_NC, num_subcores=_NS),
    compiler_params=pltpu.CompilerParams(needs_layout_passes=False),
    scratch_types=[
        pltpu.VMEM((NUM_EXPERTS, RPW), jnp.float32),
        pltpu.VMEM((RPW * NUM_EXPERTS,), jnp.float32),
        pltpu.VMEM((NUM_EXPERTS * 16,), jnp.float32),
    ],
)


@jax.jit
def kernel(g_emb, W, b, alpha, expert_biases):
    b2 = b.reshape(NUM_EXPERTS, 1)
    eb2 = expert_biases.reshape(NUM_EXPERTS, 1)
    alpha1 = alpha.reshape(1)
    ltT = _tc_logits(g_emb, W, b2, alpha1, eb2)
    return _route_sc(ltT).reshape(TOKENS, NUM_EXPERTS)


# speedup vs baseline: 1.0853x; 1.0853x over previous
import functools

import jax
import jax.numpy as jnp
from jax import lax
from jax.experimental import pallas as pl
from jax.experimental.pallas import tpu as pltpu
from jax.experimental.pallas import tpu_sc as plsc

TOKENS = 8192
DIM = 2048
NUM_EXPERTS = 64
TOP_K = 8
TEMPERATURE = 0.5
BLOCK_T = 1024

_NC = 2
_NS = 16
_NW = _NC * _NS
RPW = TOKENS // _NW
NG = RPW // 16

_SORT8 = [(0, 1), (2, 3), (4, 5), (6, 7),
          (0, 2), (1, 3), (4, 6), (5, 7),
          (1, 2), (5, 6), (0, 4), (3, 7),
          (1, 5), (2, 6),
          (1, 4), (3, 6),
          (2, 4), (3, 5),
          (3, 4)]
_BITONIC8 = [(0, 4), (1, 5), (2, 6), (3, 7),
             (0, 2), (1, 3), (4, 6), (5, 7),
             (0, 1), (2, 3), (4, 5), (6, 7)]


def _exp_block(g_ref, w_ref, b_ref, alpha_ref, eb_ref, out_ref):
    w = w_ref[...]
    g = g_ref[...]
    scale = alpha_ref[0] / TEMPERATURE
    mm = lax.dot_general(w, g, (((1,), (1,)), ((), ())),
                         preferred_element_type=jnp.float32)
    lt = (mm + b_ref[...]) * scale + eb_ref[...]
    mx = jnp.max(lt, axis=0, keepdims=True)
    out_ref[...] = jnp.exp(lt - mx)


def _tc_exp(g_emb, W, b2, alpha1, eb2):
    return pl.pallas_call(
        _exp_block,
        grid=(TOKENS // BLOCK_T,),
        in_specs=[
            pl.BlockSpec((BLOCK_T, DIM), lambda i: (i, 0)),
            pl.BlockSpec((NUM_EXPERTS, DIM), lambda i: (0, 0)),
            pl.BlockSpec((NUM_EXPERTS, 1), lambda i: (0, 0)),
            pl.BlockSpec(memory_space=pltpu.SMEM),
            pl.BlockSpec((NUM_EXPERTS, 1), lambda i: (0, 0)),
        ],
        out_specs=pl.BlockSpec((NUM_EXPERTS, BLOCK_T), lambda i: (0, i)),
        out_shape=jax.ShapeDtypeStruct((NUM_EXPERTS, TOKENS), jnp.float32),
    )(g_emb, W, b2, alpha1, eb2)


def _ce_desc(lst, i, j):
    hi = jnp.maximum(lst[i], lst[j])
    lo = jnp.minimum(lst[i], lst[j])
    lst[i] = hi
    lst[j] = lo


def _merge_top8(a, b, clean):
    t = [jnp.maximum(a[i], b[7 - i]) for i in range(8)]
    if clean:
        for i, j in _BITONIC8:
            _ce_desc(t, i, j)
    return t


def _splat(vec, lane):
    idx = jnp.full((16,), lane, jnp.int32)
    return vec.at[idx].get(mode="promise_in_bounds")


def _route_body(et_hbm, out_hbm, in_v, out_v):
    wid = lax.axis_index("s") * _NC + lax.axis_index("c")
    base = wid * RPW
    pltpu.sync_copy(et_hbm.at[:, pl.ds(base, RPW)], in_v)

    def group(gi, carry):
        col = gi * 16
        v = [in_v[e, pl.ds(col, 16)] for e in range(NUM_EXPERTS)]

        sgs = []
        for g8 in range(8):
            grp = [v[g8 * 8 + i] for i in range(8)]
            for i, j in _SORT8:
                _ce_desc(grp, i, j)
            sgs.append(grp)
        l1 = [_merge_top8(sgs[2 * i], sgs[2 * i + 1], True) for i in range(4)]
        l2 = [_merge_top8(l1[0], l1[1], True), _merge_top8(l1[2], l1[3], True)]
        t = _merge_top8(l2[0], l2[1], False)
        thr = t[0]
        for i in range(1, 8):
            thr = jnp.minimum(thr, t[i])

        acc_all = [jnp.zeros((16,), jnp.float32) for _ in range(4)]
        acc_sel = [jnp.zeros((16,), jnp.float32) for _ in range(4)]
        for e in range(NUM_EXPERTS):
            ev = in_v[e, pl.ds(col, 16)]
            acc_all[e % 4] = acc_all[e % 4] + ev
            acc_sel[e % 4] = acc_sel[e % 4] + jnp.where(ev >= thr, ev, 0.0)
        s_all = (acc_all[0] + acc_all[1]) + (acc_all[2] + acc_all[3])
        s_sel = (acc_sel[0] + acc_sel[1]) + (acc_sel[2] + acc_sel[3])
        inv = 1.0 / (s_sel + 1e-12 * s_all)

        stride = lax.iota(jnp.int32, 16) * RPW
        for i in range(16):
            thr_t = _splat(thr, i)
            inv_t = _splat(inv, i)
            for j in range(4):
                vt = plsc.load_gather(
                    in_v, [jnp.full((16,), 16 * j, jnp.int32)
                           + lax.iota(jnp.int32, 16),
                           jnp.full((16,), col + i, jnp.int32)])
                w = jnp.where(vt >= thr_t, vt * inv_t, 0.0)
                out_v[pl.ds((col + i) * NUM_EXPERTS + 16 * j, 16)] = w
        return carry

    lax.fori_loop(0, NG, group, 0)
    pltpu.sync_copy(out_v, out_hbm.at[pl.ds(base * NUM_EXPERTS, RPW * NUM_EXPERTS)])


_route_sc = pl.kernel(
    _route_body,
    out_type=jax.ShapeDtypeStruct((TOKENS * NUM_EXPERTS,), jnp.float32),
    mesh=plsc.VectorSubcoreMesh(core_axis_name="c", subcore_axis_name="s",
                                num_cores=_NC, num_subcores=_NS),
    compiler_params=pltpu.CompilerParams(needs_layout_passes=False),
    scratch_types=[
        pltpu.VMEM((NUM_EXPERTS, RPW), jnp.float32),
        pltpu.VMEM((RPW * NUM_EXPERTS,), jnp.float32),
    ],
)


@jax.jit
def kernel(g_emb, W, b, alpha, expert_biases):
    b2 = b.reshape(NUM_EXPERTS, 1)
    eb2 = expert_biases.reshape(NUM_EXPERTS, 1)
    alpha1 = alpha.reshape(1)
    etT = _tc_exp(g_emb, W, b2, alpha1, eb2)
    return _route_sc(etT).reshape(TOKENS, NUM_EXPERTS)


# speedup vs baseline: 1.8501x; 1.7047x over previous
import functools

import jax
import jax.numpy as jnp
from jax.experimental import pallas as pl
from jax.experimental.pallas import tpu as pltpu

TOKENS = 8192
DIM = 2048
NUM_EXPERTS = 64
TOP_K = 8
TEMPERATURE = 0.5
BLOCK_T = 512


SUB = 4


def _gating_block(g_ref, wt_ref, b_ref, alpha_ref, eb_ref, out_ref):
    wt = wt_ref[...]
    scale = alpha_ref[0] / TEMPERATURE
    sub_t = BLOCK_T // SUB
    for s in range(SUB):
        rows = pl.ds(s * sub_t, sub_t)
        g = g_ref[rows, :]
        base = jax.lax.dot_general(
            g, wt, (((1,), (0,)), ((), ())),
            preferred_element_type=jnp.float32)
        logits = (base + b_ref[...]) * scale + eb_ref[...]

        cur = logits
        mask = jnp.zeros(logits.shape, jnp.float32)
        neg_inf = jnp.float32(-jnp.inf)
        for _ in range(TOP_K):
            m = jnp.max(cur, axis=1, keepdims=True)
            sel = cur == m
            mask = jnp.where(sel, 1.0, mask)
            cur = jnp.where(sel, neg_inf, cur)

        mx = jnp.max(logits, axis=1, keepdims=True)
        e = jnp.exp(logits - mx)
        em = e * mask
        denom = (jnp.sum(em, axis=1, keepdims=True)
                 + 1e-12 * jnp.sum(e, axis=1, keepdims=True))
        out_ref[rows, :] = em / denom


@jax.jit
def kernel(g_emb, W, b, alpha, expert_biases):
    wt = W.T
    b2 = b.reshape(1, NUM_EXPERTS)
    eb2 = expert_biases.reshape(1, NUM_EXPERTS)
    alpha1 = alpha.reshape(1)
    grid = (TOKENS // BLOCK_T,)
    return pl.pallas_call(
        _gating_block,
        grid=grid,
        in_specs=[
            pl.BlockSpec((BLOCK_T, DIM), lambda i: (i, 0)),
            pl.BlockSpec((DIM, NUM_EXPERTS), lambda i: (0, 0)),
            pl.BlockSpec((1, NUM_EXPERTS), lambda i: (0, 0)),
            pl.BlockSpec(memory_space=pltpu.SMEM),
            pl.BlockSpec((1, NUM_EXPERTS), lambda i: (0, 0)),
        ],
        out_specs=pl.BlockSpec((BLOCK_T, NUM_EXPERTS), lambda i: (i, 0)),
        out_shape=jax.ShapeDtypeStruct((TOKENS, NUM_EXPERTS), jnp.float32),
    )(g_emb, wt, b2, alpha1, eb2)


# speedup vs baseline: 2.1879x; 1.1826x over previous
import functools

import jax
import jax.numpy as jnp
from jax.experimental import pallas as pl
from jax.experimental.pallas import tpu as pltpu

TOKENS = 8192
DIM = 2048
NUM_EXPERTS = 64
TOP_K = 8
TEMPERATURE = 0.5
BLOCK_T = 2048


SUB = 4


def _gating_block(g_ref, wt_ref, b_ref, alpha_ref, eb_ref, out_ref):
    wt = wt_ref[...]
    scale = alpha_ref[0] / TEMPERATURE
    sub_t = BLOCK_T // SUB
    for s in range(SUB):
        rows = pl.ds(s * sub_t, sub_t)
        g = g_ref[rows, :]
        base = jax.lax.dot_general(
            g, wt, (((1,), (0,)), ((), ())),
            preferred_element_type=jnp.float32)
        logits = (base + b_ref[...]) * scale + eb_ref[...]

        cur = logits
        mask = jnp.zeros(logits.shape, jnp.float32)
        neg_inf = jnp.float32(-jnp.inf)
        for _ in range(TOP_K):
            m = jnp.max(cur, axis=1, keepdims=True)
            sel = cur == m
            mask = jnp.where(sel, 1.0, mask)
            cur = jnp.where(sel, neg_inf, cur)

        mx = jnp.max(logits, axis=1, keepdims=True)
        e = jnp.exp(logits - mx)
        em = e * mask
        denom = (jnp.sum(em, axis=1, keepdims=True)
                 + 1e-12 * jnp.sum(e, axis=1, keepdims=True))
        out_ref[rows, :] = em / denom


@jax.jit
def kernel(g_emb, W, b, alpha, expert_biases):
    wt = W.T
    b2 = b.reshape(1, NUM_EXPERTS)
    eb2 = expert_biases.reshape(1, NUM_EXPERTS)
    alpha1 = alpha.reshape(1)
    grid = (TOKENS // BLOCK_T,)
    return pl.pallas_call(
        _gating_block,
        grid=grid,
        in_specs=[
            pl.BlockSpec((BLOCK_T, DIM), lambda i: (i, 0)),
            pl.BlockSpec((DIM, NUM_EXPERTS), lambda i: (0, 0)),
            pl.BlockSpec((1, NUM_EXPERTS), lambda i: (0, 0)),
            pl.BlockSpec(memory_space=pltpu.SMEM),
            pl.BlockSpec((1, NUM_EXPERTS), lambda i: (0, 0)),
        ],
        out_specs=pl.BlockSpec((BLOCK_T, NUM_EXPERTS), lambda i: (i, 0)),
        out_shape=jax.ShapeDtypeStruct((TOKENS, NUM_EXPERTS), jnp.float32),
    )(g_emb, wt, b2, alpha1, eb2)
